# baseline (speedup 1.0000x reference)
import functools

import jax
import jax.numpy as jnp
from jax import lax
from jax.experimental import pallas as pl
from jax.experimental.pallas import tpu as pltpu
from jax.experimental.pallas import tpu_sc as plsc

_NC = 2
_NS = 16
_LANES = 16
_K = 128
_NBUF = 3


def _deg_sc(row3, w3, n_node):
    nb = row3.shape[1]
    mesh = plsc.VectorSubcoreMesh(core_axis_name="c", subcore_axis_name="s")

    @functools.partial(
        pl.kernel,
        out_type=jax.ShapeDtypeStruct((_NC * _NS, n_node), jnp.float32),
        mesh=mesh,
        compiler_params=pltpu.CompilerParams(needs_layout_passes=False),
        scratch_types=[
            pltpu.VMEM((n_node,), jnp.float32),
            pltpu.VMEM((nb, _K), jnp.int32),
            pltpu.VMEM((nb, _K), jnp.float32),
        ],
    )
    def k(row_hbm, w_hbm, out_hbm, dacc, ridx, wv):
        c = lax.axis_index("c")
        s = lax.axis_index("s")
        wid = s * _NC + c
        pltpu.sync_copy(row_hbm.at[wid], ridx)
        pltpu.sync_copy(w_hbm.at[wid], wv)
        zeros = jnp.zeros((_LANES,), jnp.float32)

        def zbody(i, carry):
            dacc[pl.ds(i * _LANES, _LANES)] = zeros
            return carry

        lax.fori_loop(0, n_node // _LANES, zbody, 0)
        spb = _K // _LANES

        def body(i, carry):
            b = i // spb
            j = i % spb
            idx = ridx[b, pl.ds(j * _LANES, _LANES)]
            w = wv[b, pl.ds(j * _LANES, _LANES)]
            plsc.addupdate_scatter(dacc, [idx], w)
            return carry

        lax.fori_loop(0, nb * spb, body, 0)
        pltpu.sync_copy(dacc, out_hbm.at[wid])

    return k(row3, w3)


def _spmm_sc(u2, col3, row3, w3, n_node, d):
    nb = col3.shape[1]
    nbr = nb // _NBUF
    dh = d // _NC
    nps = n_node // _NS
    mesh = plsc.VectorSubcoreMesh(core_axis_name="c", subcore_axis_name="s")

    @functools.partial(
        pl.kernel,
        out_type=jax.ShapeDtypeStruct((_NC, n_node, dh), jnp.float32),
        mesh=mesh,
        compiler_params=pltpu.CompilerParams(
            needs_layout_passes=False, use_tc_tiling_on_sc=False),
        scratch_types=[
            pltpu.VMEM((nb, _K), jnp.int32),
            pltpu.VMEM((nb, _K), jnp.int32),
            pltpu.VMEM((nb, _K), jnp.float32),
        ] + [pltpu.VMEM((_K, dh), jnp.float32)] * _NBUF
          + [pltpu.VMEM_SHARED((n_node, dh), jnp.float32)]
          + [pltpu.SemaphoreType.DMA] * (2 * _NBUF),
    )
    def k(u_hbm, col_hbm, row_hbm, w_hbm, out_hbm, cidx, ridx, wv, *rest):
        bufs = rest[:_NBUF]
        acc = rest[_NBUF]
        gsems = rest[_NBUF + 1:2 * _NBUF + 1]
        ssems = rest[2 * _NBUF + 1:]
        c = lax.axis_index("c")
        s = lax.axis_index("s")
        uc = u_hbm.at[c]
        pltpu.sync_copy(col_hbm.at[s], cidx)
        pltpu.sync_copy(row_hbm.at[s], ridx)
        pltpu.sync_copy(w_hbm.at[s], wv)

        zeros = jnp.zeros((_LANES,), jnp.float32)
        vpr = dh // _LANES

        def zrow(i, carry):
            bufs[0][i // vpr, pl.ds((i % vpr) * _LANES, _LANES)] = zeros
            return carry

        lax.fori_loop(0, _K * vpr, zrow, 0)
        nb0 = s * nps
        nfull = nps // _K
        for t in range(nfull):
            pltpu.sync_copy(bufs[0], acc.at[pl.ds(nb0 + t * _K, _K)])
        rem = nps - nfull * _K
        if rem:
            pltpu.sync_copy(bufs[0].at[pl.ds(0, rem)],
                            acc.at[pl.ds(nb0 + nfull * _K, rem)])
        plsc.subcore_barrier()

        def scale_buf(b, q):
            rows = bufs[q]

            def scale(g, c2):
                wvec = wv[b, pl.ds(g * _LANES, _LANES)]
                for t in range(_LANES):
                    e = g * _LANES + t
                    w = wvec[t]
                    for j in range(vpr):
                        rows[e, pl.ds(j * _LANES, _LANES)] = (
                            rows[e, pl.ds(j * _LANES, _LANES)] * w)
                return c2

            lax.fori_loop(0, _K // _LANES, scale, 0)

        for q in range(_NBUF):
            pltpu.async_copy(uc.at[cidx.at[q]], bufs[q], gsems[q])

        def round_body(i, carry):
            b0 = i * _NBUF
            sdescs = []
            for q in range(_NBUF):
                b = b0 + q
                pltpu.make_async_copy(
                    uc.at[cidx.at[b]], bufs[q], gsems[q]).wait()
                scale_buf(b, q)
                sdescs.append(pltpu.async_copy(
                    bufs[q], acc.at[ridx.at[b]], ssems[q], add=True))
            for q in range(_NBUF):
                sdescs[q].wait()

                @pl.when(i < nbr - 1)
                def _():
                    pltpu.async_copy(
                        uc.at[cidx.at[b0 + _NBUF + q]], bufs[q], gsems[q])
            return carry

        lax.fori_loop(0, nbr, round_body, 0)
        plsc.subcore_barrier()
        pltpu.sync_copy(acc.at[pl.ds(nb0, nps)],
                        out_hbm.at[c, pl.ds(nb0, nps)])

    return k(u2, col3, row3, w3)


def _prep_tc(dp_t, x, araw_page):
    n, d = x.shape
    dh = d // _NC

    def body(dp_ref, x_ref, a_ref, u_ref, dinv_ref, al_ref):
        deg = jnp.sum(dp_ref[...], axis=1, keepdims=True)
        pos = deg > 0.0
        dinv = jnp.where(pos, lax.rsqrt(jnp.where(pos, deg, 1.0)), 0.0)
        dinv_ref[...] = dinv
        u = dinv * x_ref[...]
        u_ref[0] = u[:, :dh]
        u_ref[1] = u[:, dh:]
        al_ref[...] = jnp.tanh(a_ref[...])

    return pl.pallas_call(
        body,
        out_shape=(
            jax.ShapeDtypeStruct((_NC, n, dh), jnp.float32),
            jax.ShapeDtypeStruct((n, 1), jnp.float32),
            jax.ShapeDtypeStruct(araw_page.shape, jnp.float32),
        ),
    )(dp_t, x, araw_page)


def _combine_tc(alpha_page, p, dinv):
    _, n, dh = p.shape

    def body(a_ref, p_ref, di_ref, xl_ref, ul_ref):
        di = di_ref[...]
        a = a_ref[:, :dh]
        lo = di * p_ref[0]
        hi = di * p_ref[1]
        xl_ref[:, :dh] = a * lo
        xl_ref[:, dh:] = a * hi
        ul_ref[0] = a * di * lo
        ul_ref[1] = a * di * hi

    return pl.pallas_call(
        body,
        out_shape=(
            jax.ShapeDtypeStruct((n, _NC * dh), jnp.float32),
            jax.ShapeDtypeStruct((_NC, n, dh), jnp.float32),
        ),
    )(alpha_page, p, dinv)


def kernel(x, edge_index, edge_attr, alphas_raw):
    n, d = x.shape
    e = edge_index.shape[1]
    depth = alphas_raw.shape[0] - 1

    nps = -(-n // _NS)
    nps = -(-nps // 8) * 8
    n_pad = nps * _NS
    xp = jnp.pad(x, ((0, n_pad - n), (0, 0)))

    blk = _K * _NBUF * _NC
    ept = -(-e // (_NS * blk)) * blk
    e_pad = ept * _NS
    pad = e_pad - e
    row = jnp.concatenate([edge_index[0], jnp.zeros((pad,), jnp.int32)])
    col = jnp.concatenate([edge_index[1], jnp.zeros((pad,), jnp.int32)])
    w = jnp.concatenate([edge_attr.astype(jnp.float32),
                         jnp.zeros((pad,), jnp.float32)])
    nb = ept // _K
    row3 = row.reshape(_NS, nb, _K)
    col3 = col.reshape(_NS, nb, _K)
    w3 = w.reshape(_NS, nb, _K)

    eptd = -(-e // (_NC * _NS * _K)) * _K
    padd = eptd * _NC * _NS - e
    rowd = jnp.concatenate([edge_index[0], jnp.zeros((padd,), jnp.int32)])
    wd = jnp.concatenate([edge_attr.astype(jnp.float32),
                          jnp.zeros((padd,), jnp.float32)])
    nbd = eptd // _K
    dp = _deg_sc(rowd.reshape(_NC * _NS, nbd, _K),
                 wd.reshape(_NC * _NS, nbd, _K), n_pad)
    araw_page = jnp.zeros((1, d), jnp.float32).at[0, :depth + 1].set(alphas_raw)
    u2, dinv, alphas = _prep_tc(dp.T, xp, araw_page)

    xs = [x]
    for layer in range(1, depth + 1):
        p = _spmm_sc(u2, col3, row3, w3, n_pad, d)
        alpha_page = jnp.broadcast_to(alphas[0:1, layer:layer + 1], (1, d))
        xl, u2 = _combine_tc(alpha_page, p, dinv)
        xs.append(xl[:n])
    return jnp.stack(xs, axis=1)


# speedup vs baseline: 1.3834x; 1.3834x over previous
import functools

import jax
import jax.numpy as jnp
from jax import lax
from jax.experimental import pallas as pl
from jax.experimental.pallas import tpu as pltpu
from jax.experimental.pallas import tpu_sc as plsc

_NC = 2
_NS = 16
_LANES = 16
_K = 128
_NBUF = 4
_SLAB = 8


def _deg_sc(row3, w3, n_node):
    nb = row3.shape[1]
    mesh = plsc.VectorSubcoreMesh(core_axis_name="c", subcore_axis_name="s")

    @functools.partial(
        pl.kernel,
        out_type=jax.ShapeDtypeStruct((_NC * _NS, n_node), jnp.float32),
        mesh=mesh,
        compiler_params=pltpu.CompilerParams(needs_layout_passes=False),
        scratch_types=[
            pltpu.VMEM((n_node,), jnp.float32),
            pltpu.VMEM((nb, _K), jnp.int32),
            pltpu.VMEM((nb, _K), jnp.float32),
        ],
    )
    def k(row_hbm, w_hbm, out_hbm, dacc, ridx, wv):
        c = lax.axis_index("c")
        s = lax.axis_index("s")
        wid = s * _NC + c
        pltpu.sync_copy(row_hbm.at[wid], ridx)
        pltpu.sync_copy(w_hbm.at[wid], wv)
        zeros = jnp.zeros((_LANES,), jnp.float32)

        def zbody(i, carry):
            dacc[pl.ds(i * _LANES, _LANES)] = zeros
            return carry

        lax.fori_loop(0, n_node // _LANES, zbody, 0)
        spb = _K // _LANES

        def body(i, carry):
            b = i // spb
            j = i % spb
            idx = ridx[b, pl.ds(j * _LANES, _LANES)]
            w = wv[b, pl.ds(j * _LANES, _LANES)]
            plsc.addupdate_scatter(dacc, [idx], w)
            return carry

        lax.fori_loop(0, nb * spb, body, 0)
        pltpu.sync_copy(dacc, out_hbm.at[wid])

    return k(row3, w3)


def _spmm_sc(u2, col3, row3, w3, n_node, d):
    nb = col3.shape[1]
    assert nb % _SLAB == 0
    nsl = nb // _SLAB
    dh = d // _NC
    nps = n_node // _NS
    mesh = plsc.VectorSubcoreMesh(core_axis_name="c", subcore_axis_name="s")

    @functools.partial(
        pl.kernel,
        out_type=jax.ShapeDtypeStruct((_NC, n_node, dh), jnp.float32),
        mesh=mesh,
        compiler_params=pltpu.CompilerParams(
            needs_layout_passes=False, use_tc_tiling_on_sc=False),
        scratch_types=[
            pltpu.VMEM((_SLAB, _K), jnp.int32),
            pltpu.VMEM((_SLAB, _K), jnp.int32),
            pltpu.VMEM((_SLAB, _K), jnp.float32),
        ] + [pltpu.VMEM((_K, dh), jnp.float32)] * _NBUF
          + [pltpu.VMEM_SHARED((n_node, dh), jnp.float32)]
          + [pltpu.VMEM_SHARED((n_node, dh), jnp.float32)]
          + [pltpu.SemaphoreType.DMA] * (2 * _NBUF),
    )
    def k(u_hbm, col_hbm, row_hbm, w_hbm, out_hbm, cslab, rslab, wslab, *rest):
        bufs = rest[:_NBUF]
        utab = rest[_NBUF]
        acc = rest[_NBUF + 1]
        gsems = rest[_NBUF + 2:2 * _NBUF + 2]
        ssems = rest[2 * _NBUF + 2:]
        c = lax.axis_index("c")
        s = lax.axis_index("s")

        nb0 = s * nps
        pltpu.sync_copy(u_hbm.at[c, pl.ds(nb0, nps)], utab.at[pl.ds(nb0, nps)])
        zeros = jnp.zeros((_LANES,), jnp.float32)
        vpr = dh // _LANES

        def zrow(i, carry):
            bufs[0][i // vpr, pl.ds((i % vpr) * _LANES, _LANES)] = zeros
            return carry

        lax.fori_loop(0, _K * vpr, zrow, 0)
        nfull = nps // _K
        for t in range(nfull):
            pltpu.sync_copy(bufs[0], acc.at[pl.ds(nb0 + t * _K, _K)])
        rem = nps - nfull * _K
        if rem:
            pltpu.sync_copy(bufs[0].at[pl.ds(0, rem)],
                            acc.at[pl.ds(nb0 + nfull * _K, rem)])
        plsc.subcore_barrier()

        def scale_buf(sr, q):
            rows = bufs[q]

            def scale(g, c2):
                wvec = wslab[sr, pl.ds(g * _LANES, _LANES)]
                for t in range(_LANES):
                    e = g * _LANES + t
                    wb = wvec.at[jnp.full((_LANES,), t, jnp.int32)].get(
                        mode="promise_in_bounds")
                    for j in range(vpr):
                        rows[e, pl.ds(j * _LANES, _LANES)] = (
                            rows[e, pl.ds(j * _LANES, _LANES)] * wb)
                return c2

            lax.fori_loop(0, _K // _LANES, scale, 0)

        def slab_body(t, carry):
            off = pl.multiple_of(t * _SLAB, 8)
            pltpu.sync_copy(col_hbm.at[s, pl.ds(off, _SLAB)], cslab)
            pltpu.sync_copy(row_hbm.at[s, pl.ds(off, _SLAB)], rslab)
            pltpu.sync_copy(w_hbm.at[s, pl.ds(off, _SLAB)], wslab)
            for half in range(_SLAB // _NBUF):
                gdescs = []
                for q in range(_NBUF):
                    sr = half * _NBUF + q
                    gdescs.append(pltpu.async_copy(
                        utab.at[cslab.at[sr]], bufs[q], gsems[q]))
                sdescs = []
                for q in range(_NBUF):
                    sr = half * _NBUF + q
                    gdescs[q].wait()
                    scale_buf(sr, q)
                    sdescs.append(pltpu.async_copy(
                        bufs[q], acc.at[rslab.at[sr]], ssems[q], add=True))
                for q in range(_NBUF):
                    sdescs[q].wait()
            return carry

        lax.fori_loop(0, nsl, slab_body, 0)
        plsc.subcore_barrier()
        pltpu.sync_copy(acc.at[pl.ds(nb0, nps)],
                        out_hbm.at[c, pl.ds(nb0, nps)])

    return k(u2, col3, row3, w3)


def _prep_tc(dp_t, x, araw_page):
    n, d = x.shape
    dh = d // _NC

    def body(dp_ref, x_ref, a_ref, u_ref, dinv_ref, al_ref):
        deg = jnp.sum(dp_ref[...], axis=1, keepdims=True)
        pos = deg > 0.0
        dinv = jnp.where(pos, lax.rsqrt(jnp.where(pos, deg, 1.0)), 0.0)
        dinv_ref[...] = dinv
        u = dinv * x_ref[...]
        u_ref[0] = u[:, :dh]
        u_ref[1] = u[:, dh:]
        al_ref[...] = jnp.tanh(a_ref[...])

    return pl.pallas_call(
        body,
        out_shape=(
            jax.ShapeDtypeStruct((_NC, n, dh), jnp.float32),
            jax.ShapeDtypeStruct((n, 1), jnp.float32),
            jax.ShapeDtypeStruct(araw_page.shape, jnp.float32),
        ),
    )(dp_t, x, araw_page)


def _combine_tc(alpha_page, p, dinv):
    _, n, dh = p.shape

    def body(a_ref, p_ref, di_ref, xl_ref, ul_ref):
        di = di_ref[...]
        a = a_ref[:, :dh]
        lo = di * p_ref[0]
        hi = di * p_ref[1]
        xl_ref[:, :dh] = a * lo
        xl_ref[:, dh:] = a * hi
        ul_ref[0] = a * di * lo
        ul_ref[1] = a * di * hi

    return pl.pallas_call(
        body,
        out_shape=(
            jax.ShapeDtypeStruct((n, _NC * dh), jnp.float32),
            jax.ShapeDtypeStruct((_NC, n, dh), jnp.float32),
        ),
    )(alpha_page, p, dinv)


def kernel(x, edge_index, edge_attr, alphas_raw):
    n, d = x.shape
    e = edge_index.shape[1]
    depth = alphas_raw.shape[0] - 1

    nps = -(-n // _NS)
    nps = -(-nps // 8) * 8
    n_pad = nps * _NS
    xp = jnp.pad(x, ((0, n_pad - n), (0, 0)))

    blk = _K * _SLAB
    ept = -(-e // (_NS * blk)) * blk
    e_pad = ept * _NS
    pad = e_pad - e
    row = jnp.concatenate([edge_index[0], jnp.zeros((pad,), jnp.int32)])
    col = jnp.concatenate([edge_index[1], jnp.zeros((pad,), jnp.int32)])
    w = jnp.concatenate([edge_attr.astype(jnp.float32),
                         jnp.zeros((pad,), jnp.float32)])
    nb = ept // _K
    row3 = row.reshape(_NS, nb, _K)
    col3 = col.reshape(_NS, nb, _K)
    w3 = w.reshape(_NS, nb, _K)

    eptd = -(-e // (_NC * _NS * _K)) * _K
    padd = eptd * _NC * _NS - e
    rowd = jnp.concatenate([edge_index[0], jnp.zeros((padd,), jnp.int32)])
    wd = jnp.concatenate([edge_attr.astype(jnp.float32),
                          jnp.zeros((padd,), jnp.float32)])
    nbd = eptd // _K
    dp = _deg_sc(rowd.reshape(_NC * _NS, nbd, _K),
                 wd.reshape(_NC * _NS, nbd, _K), n_pad)
    araw_page = jnp.zeros((1, d), jnp.float32).at[0, :depth + 1].set(alphas_raw)
    u2, dinv, alphas = _prep_tc(dp.T, xp, araw_page)

    xs = [x]
    for layer in range(1, depth + 1):
        p = _spmm_sc(u2, col3, row3, w3, n_pad, d)
        alpha_page = jnp.broadcast_to(alphas[0:1, layer:layer + 1], (1, d))
        xl, u2 = _combine_tc(alpha_page, p, dinv)
        xs.append(xl[:n])
    return jnp.stack(xs, axis=1)


# speedup vs baseline: 2.4319x; 1.7579x over previous
import functools

import jax
import jax.numpy as jnp
from jax import lax
from jax.experimental import pallas as pl
from jax.experimental.pallas import tpu as pltpu
from jax.experimental.pallas import tpu_sc as plsc

_NC = 2
_NS = 16
_LANES = 16
_K = 128
_NBUF = 4
_SLAB = 8


def _deg_sc(row3, w3, n_node):
    nb = row3.shape[1]
    mesh = plsc.VectorSubcoreMesh(core_axis_name="c", subcore_axis_name="s")

    @functools.partial(
        pl.kernel,
        out_type=jax.ShapeDtypeStruct((_NC * _NS, n_node), jnp.float32),
        mesh=mesh,
        compiler_params=pltpu.CompilerParams(needs_layout_passes=False),
        scratch_types=[
            pltpu.VMEM((n_node,), jnp.float32),
            pltpu.VMEM((nb, _K), jnp.int32),
            pltpu.VMEM((nb, _K), jnp.float32),
        ],
    )
    def k(row_hbm, w_hbm, out_hbm, dacc, ridx, wv):
        c = lax.axis_index("c")
        s = lax.axis_index("s")
        wid = s * _NC + c
        pltpu.sync_copy(row_hbm.at[wid], ridx)
        pltpu.sync_copy(w_hbm.at[wid], wv)
        zeros = jnp.zeros((_LANES,), jnp.float32)

        def zbody(i, carry):
            dacc[pl.ds(i * _LANES, _LANES)] = zeros
            return carry

        lax.fori_loop(0, n_node // _LANES, zbody, 0)
        spb = _K // _LANES

        def body(i, carry):
            b = i // spb
            j = i % spb
            idx = ridx[b, pl.ds(j * _LANES, _LANES)]
            w = wv[b, pl.ds(j * _LANES, _LANES)]
            plsc.addupdate_scatter(dacc, [idx], w)
            return carry

        lax.fori_loop(0, nb * spb, body, 0)
        pltpu.sync_copy(dacc, out_hbm.at[wid])

    return k(row3, w3)


def _spmm_sc(u2, col3, row3, w3, n_node, d):
    nb = col3.shape[1]
    assert nb % _SLAB == 0
    nsl = nb // _SLAB
    dh = d // _NC
    nps = n_node // _NS
    mesh = plsc.VectorSubcoreMesh(core_axis_name="c", subcore_axis_name="s")

    @functools.partial(
        pl.kernel,
        out_type=jax.ShapeDtypeStruct((_NC, n_node, dh), jnp.float32),
        mesh=mesh,
        compiler_params=pltpu.CompilerParams(
            needs_layout_passes=False, use_tc_tiling_on_sc=False),
        scratch_types=[
            pltpu.VMEM((_SLAB, _K), jnp.int32),
            pltpu.VMEM((_SLAB, _K), jnp.int32),
            pltpu.VMEM((_SLAB, _K), jnp.float32),
        ] + [pltpu.VMEM((_K, dh), jnp.float32)] * _NBUF
          + [pltpu.VMEM_SHARED((n_node, dh), jnp.float32)]
          + [pltpu.VMEM_SHARED((n_node, dh), jnp.float32)]
          + [pltpu.SemaphoreType.DMA] * (2 * _NBUF),
    )
    def k(u_hbm, col_hbm, row_hbm, w_hbm, out_hbm, cslab, rslab, wslab, *rest):
        gbufs = rest[:2]
        sbufs = rest[2:4]
        utab = rest[_NBUF]
        acc = rest[_NBUF + 1]
        gsems = rest[_NBUF + 2:2 * _NBUF + 2]
        ssems = rest[2 * _NBUF + 2:]
        c = lax.axis_index("c")
        s = lax.axis_index("s")

        nb0 = s * nps
        pltpu.sync_copy(u_hbm.at[c, pl.ds(nb0, nps)], utab.at[pl.ds(nb0, nps)])
        zeros = jnp.zeros((_LANES,), jnp.float32)
        vpr = dh // _LANES

        def zrow(i, carry):
            gbufs[0][i // vpr, pl.ds((i % vpr) * _LANES, _LANES)] = zeros
            return carry

        lax.fori_loop(0, _K * vpr, zrow, 0)
        nfull = nps // _K
        for t in range(nfull):
            pltpu.sync_copy(gbufs[0], acc.at[pl.ds(nb0 + t * _K, _K)])
        rem = nps - nfull * _K
        if rem:
            pltpu.sync_copy(gbufs[0].at[pl.ds(0, rem)],
                            acc.at[pl.ds(nb0 + nfull * _K, rem)])
        plsc.subcore_barrier()

        def scale_buf(sr, p):
            gbuf, sbuf = gbufs[p], sbufs[p]

            def scale(g):
                wvec = wslab[sr, pl.ds(g * _LANES, _LANES)]
                for t in range(_LANES):
                    e = g * _LANES + t
                    w = wvec[t]
                    for j in range(vpr):
                        sbuf[e, pl.ds(j * _LANES, _LANES)] = (
                            gbuf[e, pl.ds(j * _LANES, _LANES)] * w)

            plsc.parallel_loop(0, _K // _LANES, unroll=2)(scale)

        def slab_body(t, carry):
            off = pl.multiple_of(t * _SLAB, 8)
            pltpu.sync_copy(col_hbm.at[s, pl.ds(off, _SLAB)], cslab)
            pltpu.sync_copy(row_hbm.at[s, pl.ds(off, _SLAB)], rslab)
            pltpu.sync_copy(w_hbm.at[s, pl.ds(off, _SLAB)], wslab)
            gdescs = [None, None]
            sdescs = [None, None]
            for b in range(2):
                gdescs[b] = pltpu.async_copy(
                    utab.at[cslab.at[b]], gbufs[b], gsems[b])
            for b in range(_SLAB):
                p = b % 2
                gdescs[p].wait()
                if b >= 2:
                    sdescs[p].wait()
                scale_buf(b, p)
                sdescs[p] = pltpu.async_copy(
                    sbufs[p], acc.at[rslab.at[b]], ssems[p], add=True)
                if b + 2 < _SLAB:
                    gdescs[p] = pltpu.async_copy(
                        utab.at[cslab.at[b + 2]], gbufs[p], gsems[p])
            sdescs[0].wait()
            sdescs[1].wait()
            return carry

        lax.fori_loop(0, nsl, slab_body, 0)
        plsc.subcore_barrier()
        pltpu.sync_copy(acc.at[pl.ds(nb0, nps)],
                        out_hbm.at[c, pl.ds(nb0, nps)])

    return k(u2, col3, row3, w3)


def _prep_tc(dp_t, x, araw_page):
    n, d = x.shape
    dh = d // _NC

    def body(dp_ref, x_ref, a_ref, u_ref, dinv_ref, al_ref):
        deg = jnp.sum(dp_ref[...], axis=1, keepdims=True)
        pos = deg > 0.0
        dinv = jnp.where(pos, lax.rsqrt(jnp.where(pos, deg, 1.0)), 0.0)
        dinv_ref[...] = dinv
        u = dinv * x_ref[...]
        u_ref[0] = u[:, :dh]
        u_ref[1] = u[:, dh:]
        al_ref[...] = jnp.tanh(a_ref[...])

    return pl.pallas_call(
        body,
        out_shape=(
            jax.ShapeDtypeStruct((_NC, n, dh), jnp.float32),
            jax.ShapeDtypeStruct((n, 1), jnp.float32),
            jax.ShapeDtypeStruct(araw_page.shape, jnp.float32),
        ),
    )(dp_t, x, araw_page)


def _combine_tc(alpha_page, p, dinv):
    _, n, dh = p.shape

    def body(a_ref, p_ref, di_ref, xl_ref, ul_ref):
        di = di_ref[...]
        a = a_ref[:, :dh]
        lo = di * p_ref[0]
        hi = di * p_ref[1]
        xl_ref[:, :dh] = a * lo
        xl_ref[:, dh:] = a * hi
        ul_ref[0] = a * di * lo
        ul_ref[1] = a * di * hi

    return pl.pallas_call(
        body,
        out_shape=(
            jax.ShapeDtypeStruct((n, _NC * dh), jnp.float32),
            jax.ShapeDtypeStruct((_NC, n, dh), jnp.float32),
        ),
    )(alpha_page, p, dinv)


def kernel(x, edge_index, edge_attr, alphas_raw):
    n, d = x.shape
    e = edge_index.shape[1]
    depth = alphas_raw.shape[0] - 1

    nps = -(-n // _NS)
    nps = -(-nps // 8) * 8
    n_pad = nps * _NS
    xp = jnp.pad(x, ((0, n_pad - n), (0, 0)))

    blk = _K * _SLAB
    ept = -(-e // (_NS * blk)) * blk
    e_pad = ept * _NS
    pad = e_pad - e
    row = jnp.concatenate([edge_index[0], jnp.zeros((pad,), jnp.int32)])
    col = jnp.concatenate([edge_index[1], jnp.zeros((pad,), jnp.int32)])
    w = jnp.concatenate([edge_attr.astype(jnp.float32),
                         jnp.zeros((pad,), jnp.float32)])
    nb = ept // _K
    row3 = row.reshape(_NS, nb, _K)
    col3 = col.reshape(_NS, nb, _K)
    w3 = w.reshape(_NS, nb, _K)

    eptd = -(-e // (_NC * _NS * _K)) * _K
    padd = eptd * _NC * _NS - e
    rowd = jnp.concatenate([edge_index[0], jnp.zeros((padd,), jnp.int32)])
    wd = jnp.concatenate([edge_attr.astype(jnp.float32),
                          jnp.zeros((padd,), jnp.float32)])
    nbd = eptd // _K
    dp = _deg_sc(rowd.reshape(_NC * _NS, nbd, _K),
                 wd.reshape(_NC * _NS, nbd, _K), n_pad)
    araw_page = jnp.zeros((1, d), jnp.float32).at[0, :depth + 1].set(alphas_raw)
    u2, dinv, alphas = _prep_tc(dp.T, xp, araw_page)

    xs = [x]
    for layer in range(1, depth + 1):
        p = _spmm_sc(u2, col3, row3, w3, n_pad, d)
        alpha_page = jnp.broadcast_to(alphas[0:1, layer:layer + 1], (1, d))
        xl, u2 = _combine_tc(alpha_page, p, dinv)
        xs.append(xl[:n])
    return jnp.stack(xs, axis=1)


# speedup vs baseline: 2.5311x; 1.0408x over previous
import functools

import jax
import jax.numpy as jnp
from jax import lax
from jax.experimental import pallas as pl
from jax.experimental.pallas import tpu as pltpu
from jax.experimental.pallas import tpu_sc as plsc

_NC = 2
_NS = 16
_LANES = 16
_K = 128
_NBUF = 4
_SLAB = 8


def _deg_sc(row3, w3, n_node):
    nb = row3.shape[1]
    mesh = plsc.VectorSubcoreMesh(core_axis_name="c", subcore_axis_name="s")

    @functools.partial(
        pl.kernel,
        out_type=jax.ShapeDtypeStruct((_NC * _NS, n_node), jnp.float32),
        mesh=mesh,
        compiler_params=pltpu.CompilerParams(needs_layout_passes=False),
        scratch_types=[
            pltpu.VMEM((n_node,), jnp.float32),
            pltpu.VMEM((nb, _K), jnp.int32),
            pltpu.VMEM((nb, _K), jnp.float32),
        ],
    )
    def k(row_hbm, w_hbm, out_hbm, dacc, ridx, wv):
        c = lax.axis_index("c")
        s = lax.axis_index("s")
        wid = s * _NC + c
        pltpu.sync_copy(row_hbm.at[wid], ridx)
        pltpu.sync_copy(w_hbm.at[wid], wv)
        zeros = jnp.zeros((_LANES,), jnp.float32)

        def zbody(i, carry):
            dacc[pl.ds(i * _LANES, _LANES)] = zeros
            return carry

        lax.fori_loop(0, n_node // _LANES, zbody, 0)
        spb = _K // _LANES

        def body(i, carry):
            b = i // spb
            j = i % spb
            idx = ridx[b, pl.ds(j * _LANES, _LANES)]
            w = wv[b, pl.ds(j * _LANES, _LANES)]
            plsc.addupdate_scatter(dacc, [idx], w)
            return carry

        lax.fori_loop(0, nb * spb, body, 0)
        pltpu.sync_copy(dacc, out_hbm.at[wid])

    return k(row3, w3)


def _poly_sc(u2, col3, row3, w3, d2, al16, n_node, d, depth):
    nb = col3.shape[1]
    assert nb % _SLAB == 0
    nsl = nb // _SLAB
    dh = d // _NC
    nps = n_node // _NS
    mesh = plsc.VectorSubcoreMesh(core_axis_name="c", subcore_axis_name="s")

    @functools.partial(
        pl.kernel,
        out_type=jax.ShapeDtypeStruct((depth, _NC, n_node, dh), jnp.float32),
        mesh=mesh,
        compiler_params=pltpu.CompilerParams(
            needs_layout_passes=False, use_tc_tiling_on_sc=False),
        scratch_types=[
            pltpu.VMEM((_SLAB, _K), jnp.int32),
            pltpu.VMEM((_SLAB, _K), jnp.int32),
            pltpu.VMEM((_SLAB, _K), jnp.float32),
            pltpu.VMEM((-(-nps // _LANES) * _LANES,), jnp.float32),
            pltpu.VMEM((_LANES,), jnp.float32),
        ] + [pltpu.VMEM((_K, dh), jnp.float32)] * _NBUF
          + [pltpu.VMEM_SHARED((n_node, dh), jnp.float32)]
          + [pltpu.VMEM_SHARED((n_node, dh), jnp.float32)]
          + [pltpu.SemaphoreType.DMA] * (2 * _NBUF),
    )
    def k(u_hbm, col_hbm, row_hbm, w_hbm, d2_hbm, al_hbm, out_hbm,
          cslab, rslab, wslab, d2v, alv, *rest):
        gbufs = rest[:2]
        sbufs = rest[2:4]
        utab = rest[_NBUF]
        acc = rest[_NBUF + 1]
        gsems = rest[_NBUF + 2:2 * _NBUF + 2]
        ssems = rest[2 * _NBUF + 2:]
        c = lax.axis_index("c")
        s = lax.axis_index("s")

        nb0 = s * nps
        pltpu.sync_copy(u_hbm.at[c, pl.ds(nb0, nps)], utab.at[pl.ds(nb0, nps)])
        pltpu.sync_copy(d2_hbm.at[pl.ds(nb0, nps)], d2v.at[pl.ds(0, nps)])
        pltpu.sync_copy(al_hbm, alv)
        zeros = jnp.zeros((_LANES,), jnp.float32)
        vpr = dh // _LANES

        def zrow(i, carry):
            gbufs[0][i // vpr, pl.ds((i % vpr) * _LANES, _LANES)] = zeros
            return carry

        nfull = nps // _K
        rem = nps - nfull * _K

        def zero_acc_slice():
            lax.fori_loop(0, _K * vpr, zrow, 0)
            for t in range(nfull):
                pltpu.sync_copy(gbufs[0], acc.at[pl.ds(nb0 + t * _K, _K)])
            if rem:
                pltpu.sync_copy(gbufs[0].at[pl.ds(0, rem)],
                                acc.at[pl.ds(nb0 + nfull * _K, rem)])

        zero_acc_slice()
        plsc.subcore_barrier()

        def scale_buf(sr, p):
            gbuf, sbuf = gbufs[p], sbufs[p]

            def scale(g):
                wvec = wslab[sr, pl.ds(g * _LANES, _LANES)]
                for t in range(_LANES):
                    e = g * _LANES + t
                    w = wvec[t]
                    for j in range(vpr):
                        sbuf[e, pl.ds(j * _LANES, _LANES)] = (
                            gbuf[e, pl.ds(j * _LANES, _LANES)] * w)

            plsc.parallel_loop(0, _K // _LANES, unroll=2)(scale)

        def slab_body(t, carry):
            off = pl.multiple_of(t * _SLAB, 8)
            pltpu.sync_copy(col_hbm.at[s, pl.ds(off, _SLAB)], cslab)
            pltpu.sync_copy(row_hbm.at[s, pl.ds(off, _SLAB)], rslab)
            pltpu.sync_copy(w_hbm.at[s, pl.ds(off, _SLAB)], wslab)
            gdescs = [None, None]
            sdescs = [None, None]
            for b in range(2):
                gdescs[b] = pltpu.async_copy(
                    utab.at[cslab.at[b]], gbufs[b], gsems[b])
            for b in range(_SLAB):
                p = b % 2
                gdescs[p].wait()
                if b >= 2:
                    sdescs[p].wait()
                scale_buf(b, p)
                sdescs[p] = pltpu.async_copy(
                    sbufs[p], acc.at[rslab.at[b]], ssems[p], add=True)
                if b + 2 < _SLAB:
                    gdescs[p] = pltpu.async_copy(
                        utab.at[cslab.at[b + 2]], gbufs[p], gsems[p])
            sdescs[0].wait()
            sdescs[1].wait()
            return carry

        avec = alv[pl.ds(0, _LANES)]

        def layer_body(ld, carry):
            lax.fori_loop(0, nsl, slab_body, 0)
            plsc.subcore_barrier()
            albc = avec.at[jnp.full((_LANES,), ld + 1, jnp.int32)].get(
                mode="promise_in_bounds")
            for t in range(nfull + (1 if rem else 0)):
                cnt = _K if t < nfull else rem
                off = nb0 + t * _K
                pltpu.sync_copy(acc.at[pl.ds(off, cnt)],
                                gbufs[0].at[pl.ds(0, cnt)])
                pltpu.sync_copy(gbufs[0].at[pl.ds(0, cnt)],
                                out_hbm.at[ld, c, pl.ds(off, cnt)])

                @pl.when(ld < depth - 1)
                def _():
                    def urow(g, c2):
                        dvec = d2v[pl.ds(t * _K + g * _LANES, _LANES)]
                        f16 = albc * dvec
                        for tt in range(_LANES):
                            r = g * _LANES + tt
                            fac = f16[tt]
                            for j in range(vpr):
                                sbufs[0][r, pl.ds(j * _LANES, _LANES)] = (
                                    gbufs[0][r, pl.ds(j * _LANES, _LANES)]
                                    * fac)
                        return c2

                    lax.fori_loop(0, -(-cnt // _LANES), urow, 0)
                    pltpu.sync_copy(sbufs[0].at[pl.ds(0, cnt)],
                                    utab.at[pl.ds(off, cnt)])

            @pl.when(ld < depth - 1)
            def _():
                zero_acc_slice()

            plsc.subcore_barrier()
            return carry

        lax.fori_loop(0, depth, layer_body, 0)

    return k(u2, col3, row3, w3, d2, al16)


def _prep_tc(dp_t, x, araw_page):
    n, d = x.shape
    dh = d // _NC

    def body(dp_ref, x_ref, a_ref, u_ref, dinv_ref, d2_ref, al_ref):
        deg = jnp.sum(dp_ref[...], axis=1, keepdims=True)
        pos = deg > 0.0
        dinv = jnp.where(pos, lax.rsqrt(jnp.where(pos, deg, 1.0)), 0.0)
        dinv_ref[...] = dinv
        d2_ref[...] = dinv * dinv
        u = dinv * x_ref[...]
        u_ref[0] = u[:, :dh]
        u_ref[1] = u[:, dh:]
        al_ref[...] = jnp.tanh(a_ref[...])

    return pl.pallas_call(
        body,
        out_shape=(
            jax.ShapeDtypeStruct((_NC, n, dh), jnp.float32),
            jax.ShapeDtypeStruct((n, 1), jnp.float32),
            jax.ShapeDtypeStruct((n, 1), jnp.float32),
            jax.ShapeDtypeStruct(araw_page.shape, jnp.float32),
        ),
    )(dp_t, x, araw_page)


def _post_tc(alphas_page, sraw, dinv):
    depth, _, n, dh = sraw.shape

    def body(a_ref, s_ref, di_ref, xs_ref):
        di = di_ref[...]
        for ld in range(depth):
            a = a_ref[0:1, ld + 1:ld + 2]
            xs_ref[ld, :, :dh] = a * (di * s_ref[ld, 0])
            xs_ref[ld, :, dh:] = a * (di * s_ref[ld, 1])

    return pl.pallas_call(
        body,
        out_shape=jax.ShapeDtypeStruct((depth, n, _NC * dh), jnp.float32),
    )(alphas_page, sraw, dinv)


def kernel(x, edge_index, edge_attr, alphas_raw):
    n, d = x.shape
    e = edge_index.shape[1]
    depth = alphas_raw.shape[0] - 1

    nps = -(-n // _NS)
    nps = -(-nps // 8) * 8
    n_pad = nps * _NS
    xp = jnp.pad(x, ((0, n_pad - n), (0, 0)))

    blk = _K * _SLAB
    ept = -(-e // (_NS * blk)) * blk
    e_pad = ept * _NS
    pad = e_pad - e
    row = jnp.concatenate([edge_index[0], jnp.zeros((pad,), jnp.int32)])
    col = jnp.concatenate([edge_index[1], jnp.zeros((pad,), jnp.int32)])
    w = jnp.concatenate([edge_attr.astype(jnp.float32),
                         jnp.zeros((pad,), jnp.float32)])
    nb = ept // _K
    row3 = row.reshape(_NS, nb, _K)
    col3 = col.reshape(_NS, nb, _K)
    w3 = w.reshape(_NS, nb, _K)

    eptd = -(-e // (_NC * _NS * _K)) * _K
    padd = eptd * _NC * _NS - e
    rowd = jnp.concatenate([edge_index[0], jnp.zeros((padd,), jnp.int32)])
    wd = jnp.concatenate([edge_attr.astype(jnp.float32),
                          jnp.zeros((padd,), jnp.float32)])
    nbd = eptd // _K
    dp = _deg_sc(rowd.reshape(_NC * _NS, nbd, _K),
                 wd.reshape(_NC * _NS, nbd, _K), n_pad)
    araw_page = jnp.zeros((1, d), jnp.float32).at[0, :depth + 1].set(alphas_raw)
    u2, dinv, dinv2, alphas = _prep_tc(dp.T, xp, araw_page)

    al16 = alphas[0, :_LANES]
    sraw = _poly_sc(u2, col3, row3, w3, dinv2.reshape(n_pad), al16,
                    n_pad, d, depth)
    xs3 = _post_tc(alphas, sraw, dinv)
    return jnp.concatenate(
        [x[:, None, :]] + [xs3[ld, :n, None, :] for ld in range(depth)],
        axis=1)


# speedup vs baseline: 2.7864x; 1.1008x over previous
import functools

import jax
import jax.numpy as jnp
from jax import lax
from jax.experimental import pallas as pl
from jax.experimental.pallas import tpu as pltpu
from jax.experimental.pallas import tpu_sc as plsc

_NC = 2
_NS = 16
_LANES = 16
_K = 128
_NBUF = 4
_SLAB = 8


def _deg_sc(row3, w3, n_node):
    nb = row3.shape[1]
    mesh = plsc.VectorSubcoreMesh(core_axis_name="c", subcore_axis_name="s")

    @functools.partial(
        pl.kernel,
        out_type=jax.ShapeDtypeStruct((_NC * _NS, n_node), jnp.float32),
        mesh=mesh,
        compiler_params=pltpu.CompilerParams(needs_layout_passes=False),
        scratch_types=[
            pltpu.VMEM((n_node,), jnp.float32),
            pltpu.VMEM((nb, _K), jnp.int32),
            pltpu.VMEM((nb, _K), jnp.float32),
        ],
    )
    def k(row_hbm, w_hbm, out_hbm, dacc, ridx, wv):
        c = lax.axis_index("c")
        s = lax.axis_index("s")
        wid = s * _NC + c
        pltpu.sync_copy(row_hbm.at[wid], ridx)
        pltpu.sync_copy(w_hbm.at[wid], wv)
        zeros = jnp.zeros((_LANES,), jnp.float32)

        def zbody(i, carry):
            dacc[pl.ds(i * _LANES, _LANES)] = zeros
            return carry

        lax.fori_loop(0, n_node // _LANES, zbody, 0)
        spb = _K // _LANES

        def body(i, carry):
            b = i // spb
            j = i % spb
            idx = ridx[b, pl.ds(j * _LANES, _LANES)]
            w = wv[b, pl.ds(j * _LANES, _LANES)]
            plsc.addupdate_scatter(dacc, [idx], w)
            return carry

        lax.fori_loop(0, nb * spb, body, 0)
        pltpu.sync_copy(dacc, out_hbm.at[wid])

    return k(row3, w3)


def _poly_sc(u2, col3, row3, w3, d2, al16, n_node, d, depth):
    nb = col3.shape[1]
    assert nb % _SLAB == 0
    nsl = nb // _SLAB
    dh = d // _NC
    nps = n_node // _NS
    mesh = plsc.VectorSubcoreMesh(core_axis_name="c", subcore_axis_name="s")

    @functools.partial(
        pl.kernel,
        out_type=jax.ShapeDtypeStruct((depth, _NC, n_node, dh), jnp.float32),
        mesh=mesh,
        compiler_params=pltpu.CompilerParams(
            needs_layout_passes=False, use_tc_tiling_on_sc=False),
        scratch_types=[
            pltpu.VMEM((2, _SLAB, _K), jnp.int32),
            pltpu.VMEM((2, _SLAB, _K), jnp.int32),
            pltpu.VMEM((2, _SLAB, _K), jnp.float32),
            pltpu.VMEM((-(-nps // _LANES) * _LANES,), jnp.float32),
            pltpu.VMEM((_LANES,), jnp.float32),
        ] + [pltpu.VMEM((_K, dh), jnp.float32)] * _NBUF
          + [pltpu.VMEM_SHARED((n_node, dh), jnp.float32)]
          + [pltpu.VMEM_SHARED((n_node, dh), jnp.float32)]
          + [pltpu.SemaphoreType.DMA] * (2 * _NBUF + 2),
    )
    def k(u_hbm, col_hbm, row_hbm, w_hbm, d2_hbm, al_hbm, out_hbm,
          cslab2, rslab2, wslab2, d2v, alv, *rest):
        gbufs = rest[:2]
        sbufs = rest[2:4]
        utab = rest[_NBUF]
        acc = rest[_NBUF + 1]
        gsems = rest[_NBUF + 2:_NBUF + 6]
        ssems = rest[_NBUF + 6:_NBUF + 10]
        psems = rest[_NBUF + 10:]
        c = lax.axis_index("c")
        s = lax.axis_index("s")

        nb0 = s * nps
        pltpu.sync_copy(u_hbm.at[c, pl.ds(nb0, nps)], utab.at[pl.ds(nb0, nps)])
        pltpu.sync_copy(d2_hbm.at[pl.ds(nb0, nps)], d2v.at[pl.ds(0, nps)])
        pltpu.sync_copy(al_hbm, alv)
        zeros = jnp.zeros((_LANES,), jnp.float32)
        vpr = dh // _LANES

        def zrow(i, carry):
            gbufs[0][i // vpr, pl.ds((i % vpr) * _LANES, _LANES)] = zeros
            return carry

        nfull = nps // _K
        rem = nps - nfull * _K

        def zero_acc_slice():
            lax.fori_loop(0, _K * vpr, zrow, 0)
            for t in range(nfull):
                pltpu.sync_copy(gbufs[0], acc.at[pl.ds(nb0 + t * _K, _K)])
            if rem:
                pltpu.sync_copy(gbufs[0].at[pl.ds(0, rem)],
                                acc.at[pl.ds(nb0 + nfull * _K, rem)])

        zero_acc_slice()
        plsc.subcore_barrier()

        def scale_buf(sl, sr, p):
            gbuf, sbuf = gbufs[p], sbufs[p]

            def scale(g):
                wvec = wslab2[sl, sr, pl.ds(g * _LANES, _LANES)]
                for t in range(_LANES):
                    e = g * _LANES + t
                    w = wvec[t]
                    for j in range(vpr):
                        sbuf[e, pl.ds(j * _LANES, _LANES)] = (
                            gbuf[e, pl.ds(j * _LANES, _LANES)] * w)

            plsc.parallel_loop(0, _K // _LANES, unroll=2)(scale)

        def load_slab(t, sl):
            off = pl.multiple_of(t * _SLAB, 8)
            pltpu.async_copy(col_hbm.at[s, pl.ds(off, _SLAB)],
                             cslab2.at[sl], psems[sl])
            pltpu.async_copy(row_hbm.at[s, pl.ds(off, _SLAB)],
                             rslab2.at[sl], psems[sl])
            pltpu.async_copy(w_hbm.at[s, pl.ds(off, _SLAB)],
                             wslab2.at[sl], psems[sl])

        def wait_slab(t, sl):
            off = pl.multiple_of(t * _SLAB, 8)
            pltpu.make_async_copy(col_hbm.at[s, pl.ds(off, _SLAB)],
                                  cslab2.at[sl], psems[sl]).wait()
            pltpu.make_async_copy(row_hbm.at[s, pl.ds(off, _SLAB)],
                                  rslab2.at[sl], psems[sl]).wait()
            pltpu.make_async_copy(w_hbm.at[s, pl.ds(off, _SLAB)],
                                  wslab2.at[sl], psems[sl]).wait()

        def slab_process(t, sl):
            cslab = cslab2.at[sl]
            rslab = rslab2.at[sl]
            gdescs = [None, None]
            sdescs = [None, None]
            for b in range(2):
                gdescs[b] = pltpu.async_copy(
                    utab.at[cslab.at[b]], gbufs[b], gsems[b])
            for b in range(_SLAB):
                p = b % 2
                gdescs[p].wait()
                if b >= 2:
                    sdescs[p].wait()
                scale_buf(sl, b, p)
                sdescs[p] = pltpu.async_copy(
                    sbufs[p], acc.at[rslab.at[b]], ssems[p], add=True)
                if b + 2 < _SLAB:
                    gdescs[p] = pltpu.async_copy(
                        utab.at[cslab.at[b + 2]], gbufs[p], gsems[p])
            sdescs[0].wait()
            sdescs[1].wait()

        def slab_pair(i2, carry):
            for sl in range(2):
                t = 2 * i2 + sl
                wait_slab(t, sl)
                slab_process(t, sl)

                @pl.when(t + 2 < nsl)
                def _():
                    load_slab(t + 2, sl)
            return carry

        avec = alv[pl.ds(0, _LANES)]

        def layer_body(ld, carry):
            load_slab(0, 0)
            load_slab(1, 1)
            lax.fori_loop(0, nsl // 2, slab_pair, 0)
            plsc.subcore_barrier()
            albc = avec.at[jnp.full((_LANES,), ld + 1, jnp.int32)].get(
                mode="promise_in_bounds")
            for t in range(nfull + (1 if rem else 0)):
                cnt = _K if t < nfull else rem
                off = nb0 + t * _K
                pltpu.sync_copy(acc.at[pl.ds(off, cnt)],
                                gbufs[0].at[pl.ds(0, cnt)])
                pltpu.sync_copy(gbufs[0].at[pl.ds(0, cnt)],
                                out_hbm.at[ld, c, pl.ds(off, cnt)])

                @pl.when(ld < depth - 1)
                def _():
                    def urow(g, c2):
                        dvec = d2v[pl.ds(t * _K + g * _LANES, _LANES)]
                        f16 = albc * dvec
                        for tt in range(_LANES):
                            r = g * _LANES + tt
                            fac = f16[tt]
                            for j in range(vpr):
                                sbufs[0][r, pl.ds(j * _LANES, _LANES)] = (
                                    gbufs[0][r, pl.ds(j * _LANES, _LANES)]
                                    * fac)
                        return c2

                    lax.fori_loop(0, -(-cnt // _LANES), urow, 0)
                    pltpu.sync_copy(sbufs[0].at[pl.ds(0, cnt)],
                                    utab.at[pl.ds(off, cnt)])

            @pl.when(ld < depth - 1)
            def _():
                zero_acc_slice()

            plsc.subcore_barrier()
            return carry

        lax.fori_loop(0, depth, layer_body, 0)

    return k(u2, col3, row3, w3, d2, al16)


def _prep_tc(dp_t, x, araw_page):
    n, d = x.shape
    dh = d // _NC

    def body(dp_ref, x_ref, a_ref, u_ref, dinv_ref, d2_ref, al_ref):
        deg = jnp.sum(dp_ref[...], axis=1, keepdims=True)
        pos = deg > 0.0
        dinv = jnp.where(pos, lax.rsqrt(jnp.where(pos, deg, 1.0)), 0.0)
        dinv_ref[...] = dinv
        d2_ref[...] = dinv * dinv
        u = dinv * x_ref[...]
        u_ref[0] = u[:, :dh]
        u_ref[1] = u[:, dh:]
        al_ref[...] = jnp.tanh(a_ref[...])

    return pl.pallas_call(
        body,
        out_shape=(
            jax.ShapeDtypeStruct((_NC, n, dh), jnp.float32),
            jax.ShapeDtypeStruct((n, 1), jnp.float32),
            jax.ShapeDtypeStruct((n, 1), jnp.float32),
            jax.ShapeDtypeStruct(araw_page.shape, jnp.float32),
        ),
    )(dp_t, x, araw_page)


def _post_tc(alphas_page, sraw, dinv):
    depth, _, n, dh = sraw.shape

    def body(a_ref, s_ref, di_ref, xs_ref):
        di = di_ref[...]
        for ld in range(depth):
            a = a_ref[0:1, ld + 1:ld + 2]
            xs_ref[ld, :, :dh] = a * (di * s_ref[ld, 0])
            xs_ref[ld, :, dh:] = a * (di * s_ref[ld, 1])

    return pl.pallas_call(
        body,
        out_shape=jax.ShapeDtypeStruct((depth, n, _NC * dh), jnp.float32),
    )(alphas_page, sraw, dinv)


def kernel(x, edge_index, edge_attr, alphas_raw):
    n, d = x.shape
    e = edge_index.shape[1]
    depth = alphas_raw.shape[0] - 1

    nps = -(-n // _NS)
    nps = -(-nps // 8) * 8
    n_pad = nps * _NS
    xp = jnp.pad(x, ((0, n_pad - n), (0, 0)))

    blk = _K * _SLAB
    ept = -(-e // (_NS * blk)) * blk
    e_pad = ept * _NS
    pad = e_pad - e
    row = jnp.concatenate([edge_index[0], jnp.zeros((pad,), jnp.int32)])
    col = jnp.concatenate([edge_index[1], jnp.zeros((pad,), jnp.int32)])
    w = jnp.concatenate([edge_attr.astype(jnp.float32),
                         jnp.zeros((pad,), jnp.float32)])
    nb = ept // _K
    row3 = row.reshape(_NS, nb, _K)
    col3 = col.reshape(_NS, nb, _K)
    w3 = w.reshape(_NS, nb, _K)

    eptd = -(-e // (_NC * _NS * _K)) * _K
    padd = eptd * _NC * _NS - e
    rowd = jnp.concatenate([edge_index[0], jnp.zeros((padd,), jnp.int32)])
    wd = jnp.concatenate([edge_attr.astype(jnp.float32),
                          jnp.zeros((padd,), jnp.float32)])
    nbd = eptd // _K
    dp = _deg_sc(rowd.reshape(_NC * _NS, nbd, _K),
                 wd.reshape(_NC * _NS, nbd, _K), n_pad)
    araw_page = jnp.zeros((1, d), jnp.float32).at[0, :depth + 1].set(alphas_raw)
    u2, dinv, dinv2, alphas = _prep_tc(dp.T, xp, araw_page)

    al16 = alphas[0, :_LANES]
    sraw = _poly_sc(u2, col3, row3, w3, dinv2.reshape(n_pad), al16,
                    n_pad, d, depth)
    xs3 = _post_tc(alphas, sraw, dinv)
    return jnp.concatenate(
        [x[:, None, :]] + [xs3[ld, :n, None, :] for ld in range(depth)],
        axis=1)


# speedup vs baseline: 2.8037x; 1.0062x over previous
import functools

import jax
import jax.numpy as jnp
from jax import lax
from jax.experimental import pallas as pl
from jax.experimental.pallas import tpu as pltpu
from jax.experimental.pallas import tpu_sc as plsc

_NC = 2
_NS = 16
_LANES = 16
_K = 128
_NBUF = 4
_SLAB = 8


def _deg_sc(row3, w3, n_node):
    nb = row3.shape[1]
    mesh = plsc.VectorSubcoreMesh(core_axis_name="c", subcore_axis_name="s")

    @functools.partial(
        pl.kernel,
        out_type=jax.ShapeDtypeStruct((_NC * _NS, n_node), jnp.float32),
        mesh=mesh,
        compiler_params=pltpu.CompilerParams(needs_layout_passes=False),
        scratch_types=[
            pltpu.VMEM((n_node,), jnp.float32),
            pltpu.VMEM((nb, _K), jnp.int32),
            pltpu.VMEM((nb, _K), jnp.float32),
        ],
    )
    def k(row_hbm, w_hbm, out_hbm, dacc, ridx, wv):
        c = lax.axis_index("c")
        s = lax.axis_index("s")
        wid = s * _NC + c
        pltpu.sync_copy(row_hbm.at[wid], ridx)
        pltpu.sync_copy(w_hbm.at[wid], wv)
        zeros = jnp.zeros((_LANES,), jnp.float32)

        def zbody(i, carry):
            dacc[pl.ds(i * _LANES, _LANES)] = zeros
            return carry

        lax.fori_loop(0, n_node // _LANES, zbody, 0)
        spb = _K // _LANES

        def body(i, carry):
            b = i // spb
            j = i % spb
            idx = ridx[b, pl.ds(j * _LANES, _LANES)]
            w = wv[b, pl.ds(j * _LANES, _LANES)]
            plsc.addupdate_scatter(dacc, [idx], w)
            return carry

        lax.fori_loop(0, nb * spb, body, 0)
        pltpu.sync_copy(dacc, out_hbm.at[wid])

    return k(row3, w3)


def _poly_sc(u2, col3, row3, w3, d2, al16, n_node, d, depth):
    nb = col3.shape[1]
    assert nb % _SLAB == 0
    nsl = nb // _SLAB
    dh = d // _NC
    nps = n_node // _NS
    mesh = plsc.VectorSubcoreMesh(core_axis_name="c", subcore_axis_name="s")

    @functools.partial(
        pl.kernel,
        out_type=jax.ShapeDtypeStruct((depth, _NC, n_node, dh), jnp.float32),
        mesh=mesh,
        compiler_params=pltpu.CompilerParams(
            needs_layout_passes=False, use_tc_tiling_on_sc=False),
        scratch_types=[
            pltpu.VMEM((2, _SLAB, _K), jnp.int32),
            pltpu.VMEM((2, _SLAB, _K), jnp.int32),
            pltpu.VMEM((2, _SLAB, _K), jnp.float32),
            pltpu.VMEM((-(-nps // _LANES) * _LANES,), jnp.float32),
            pltpu.VMEM((_LANES,), jnp.float32),
        ] + [pltpu.VMEM((_K, dh), jnp.float32)] * _NBUF
          + [pltpu.VMEM_SHARED((n_node, dh), jnp.float32)]
          + [pltpu.VMEM_SHARED((n_node, dh), jnp.float32)]
          + [pltpu.SemaphoreType.DMA] * (2 * _NBUF + 2),
    )
    def k(u_hbm, col_hbm, row_hbm, w_hbm, d2_hbm, al_hbm, out_hbm,
          cslab2, rslab2, wslab2, d2v, alv, *rest):
        gbufs = rest[:2]
        sbufs = rest[2:4]
        utab = rest[_NBUF]
        acc = rest[_NBUF + 1]
        gsems = rest[_NBUF + 2:_NBUF + 6]
        ssems = rest[_NBUF + 6:_NBUF + 10]
        psems = rest[_NBUF + 10:]
        c = lax.axis_index("c")
        s = lax.axis_index("s")

        nb0 = s * nps
        pltpu.sync_copy(u_hbm.at[c, pl.ds(nb0, nps)], utab.at[pl.ds(nb0, nps)])
        pltpu.sync_copy(d2_hbm.at[pl.ds(nb0, nps)], d2v.at[pl.ds(0, nps)])
        pltpu.sync_copy(al_hbm, alv)
        zeros = jnp.zeros((_LANES,), jnp.float32)
        vpr = dh // _LANES

        def zrow(i, carry):
            gbufs[0][i // vpr, pl.ds((i % vpr) * _LANES, _LANES)] = zeros
            return carry

        nfull = nps // _K
        rem = nps - nfull * _K

        def zero_acc_slice():
            lax.fori_loop(0, _K * vpr, zrow, 0)
            for t in range(nfull):
                pltpu.sync_copy(gbufs[0], acc.at[pl.ds(nb0 + t * _K, _K)])
            if rem:
                pltpu.sync_copy(gbufs[0].at[pl.ds(0, rem)],
                                acc.at[pl.ds(nb0 + nfull * _K, rem)])

        zero_acc_slice()
        plsc.subcore_barrier()

        def scale_buf(sl, sr, p):
            gbuf, sbuf = gbufs[p], sbufs[p]

            def scale(g):
                wvec = wslab2[sl, sr, pl.ds(g * _LANES, _LANES)]
                for t in range(_LANES):
                    e = g * _LANES + t
                    w = wvec[t]
                    for j in range(vpr):
                        sbuf[e, pl.ds(j * _LANES, _LANES)] = (
                            gbuf[e, pl.ds(j * _LANES, _LANES)] * w)

            plsc.parallel_loop(0, _K // _LANES, unroll=4)(scale)

        def load_slab(t, sl):
            off = pl.multiple_of(t * _SLAB, 8)
            pltpu.async_copy(col_hbm.at[s, pl.ds(off, _SLAB)],
                             cslab2.at[sl], psems[sl])
            pltpu.async_copy(row_hbm.at[s, pl.ds(off, _SLAB)],
                             rslab2.at[sl], psems[sl])
            pltpu.async_copy(w_hbm.at[s, pl.ds(off, _SLAB)],
                             wslab2.at[sl], psems[sl])

        def wait_slab(t, sl):
            off = pl.multiple_of(t * _SLAB, 8)
            pltpu.make_async_copy(col_hbm.at[s, pl.ds(off, _SLAB)],
                                  cslab2.at[sl], psems[sl]).wait()
            pltpu.make_async_copy(row_hbm.at[s, pl.ds(off, _SLAB)],
                                  rslab2.at[sl], psems[sl]).wait()
            pltpu.make_async_copy(w_hbm.at[s, pl.ds(off, _SLAB)],
                                  wslab2.at[sl], psems[sl]).wait()

        def slab_process(t, sl):
            cslab = cslab2.at[sl]
            rslab = rslab2.at[sl]
            gdescs = [None, None]
            sdescs = [None, None]
            for b in range(2):
                gdescs[b] = pltpu.async_copy(
                    utab.at[cslab.at[b]], gbufs[b], gsems[b])
            for b in range(_SLAB):
                p = b % 2
                gdescs[p].wait()
                if b >= 2:
                    sdescs[p].wait()
                scale_buf(sl, b, p)
                sdescs[p] = pltpu.async_copy(
                    sbufs[p], acc.at[rslab.at[b]], ssems[p], add=True)
                if b + 2 < _SLAB:
                    gdescs[p] = pltpu.async_copy(
                        utab.at[cslab.at[b + 2]], gbufs[p], gsems[p])
            sdescs[0].wait()
            sdescs[1].wait()

        def slab_pair(i2, carry):
            for sl in range(2):
                t = 2 * i2 + sl
                wait_slab(t, sl)
                slab_process(t, sl)

                @pl.when(t + 2 < nsl)
                def _():
                    load_slab(t + 2, sl)
            return carry

        avec = alv[pl.ds(0, _LANES)]

        def layer_body(ld, carry):
            load_slab(0, 0)
            load_slab(1, 1)
            lax.fori_loop(0, nsl // 2, slab_pair, 0)
            plsc.subcore_barrier()
            albc = avec.at[jnp.full((_LANES,), ld + 1, jnp.int32)].get(
                mode="promise_in_bounds")
            for t in range(nfull + (1 if rem else 0)):
                cnt = _K if t < nfull else rem
                off = nb0 + t * _K
                pltpu.sync_copy(acc.at[pl.ds(off, cnt)],
                                gbufs[0].at[pl.ds(0, cnt)])
                pltpu.sync_copy(gbufs[0].at[pl.ds(0, cnt)],
                                out_hbm.at[ld, c, pl.ds(off, cnt)])

                @pl.when(ld < depth - 1)
                def _():
                    def urow(g, c2):
                        dvec = d2v[pl.ds(t * _K + g * _LANES, _LANES)]
                        f16 = albc * dvec
                        for tt in range(_LANES):
                            r = g * _LANES + tt
                            fac = f16[tt]
                            for j in range(vpr):
                                sbufs[0][r, pl.ds(j * _LANES, _LANES)] = (
                                    gbufs[0][r, pl.ds(j * _LANES, _LANES)]
                                    * fac)
                        return c2

                    lax.fori_loop(0, -(-cnt // _LANES), urow, 0)
                    pltpu.sync_copy(sbufs[0].at[pl.ds(0, cnt)],
                                    utab.at[pl.ds(off, cnt)])

            @pl.when(ld < depth - 1)
            def _():
                zero_acc_slice()

            plsc.subcore_barrier()
            return carry

        lax.fori_loop(0, depth, layer_body, 0)

    return k(u2, col3, row3, w3, d2, al16)


def _prep_tc(dp_t, x, araw_page):
    n, d = x.shape
    dh = d // _NC

    def body(dp_ref, x_ref, a_ref, u_ref, dinv_ref, d2_ref, al_ref):
        deg = jnp.sum(dp_ref[...], axis=1, keepdims=True)
        pos = deg > 0.0
        dinv = jnp.where(pos, lax.rsqrt(jnp.where(pos, deg, 1.0)), 0.0)
        dinv_ref[...] = dinv
        d2_ref[...] = dinv * dinv
        u = dinv * x_ref[...]
        u_ref[0] = u[:, :dh]
        u_ref[1] = u[:, dh:]
        al_ref[...] = jnp.tanh(a_ref[...])

    return pl.pallas_call(
        body,
        out_shape=(
            jax.ShapeDtypeStruct((_NC, n, dh), jnp.float32),
            jax.ShapeDtypeStruct((n, 1), jnp.float32),
            jax.ShapeDtypeStruct((n, 1), jnp.float32),
            jax.ShapeDtypeStruct(araw_page.shape, jnp.float32),
        ),
    )(dp_t, x, araw_page)


def _post_tc(alphas_page, sraw, dinv):
    depth, _, n, dh = sraw.shape

    def body(a_ref, s_ref, di_ref, xs_ref):
        di = di_ref[...]
        for ld in range(depth):
            a = a_ref[0:1, ld + 1:ld + 2]
            xs_ref[ld, :, :dh] = a * (di * s_ref[ld, 0])
            xs_ref[ld, :, dh:] = a * (di * s_ref[ld, 1])

    return pl.pallas_call(
        body,
        out_shape=jax.ShapeDtypeStruct((depth, n, _NC * dh), jnp.float32),
    )(alphas_page, sraw, dinv)


def kernel(x, edge_index, edge_attr, alphas_raw):
    n, d = x.shape
    e = edge_index.shape[1]
    depth = alphas_raw.shape[0] - 1

    nps = -(-n // _NS)
    nps = -(-nps // 8) * 8
    n_pad = nps * _NS
    xp = jnp.pad(x, ((0, n_pad - n), (0, 0)))

    blk = _K * _SLAB
    ept = -(-e // (_NS * blk)) * blk
    e_pad = ept * _NS
    pad = e_pad - e
    row = jnp.concatenate([edge_index[0], jnp.zeros((pad,), jnp.int32)])
    col = jnp.concatenate([edge_index[1], jnp.zeros((pad,), jnp.int32)])
    w = jnp.concatenate([edge_attr.astype(jnp.float32),
                         jnp.zeros((pad,), jnp.float32)])
    nb = ept // _K
    row3 = row.reshape(_NS, nb, _K)
    col3 = col.reshape(_NS, nb, _K)
    w3 = w.reshape(_NS, nb, _K)

    eptd = -(-e // (_NC * _NS * _K)) * _K
    padd = eptd * _NC * _NS - e
    rowd = jnp.concatenate([edge_index[0], jnp.zeros((padd,), jnp.int32)])
    wd = jnp.concatenate([edge_attr.astype(jnp.float32),
                          jnp.zeros((padd,), jnp.float32)])
    nbd = eptd // _K
    dp = _deg_sc(rowd.reshape(_NC * _NS, nbd, _K),
                 wd.reshape(_NC * _NS, nbd, _K), n_pad)
    araw_page = jnp.zeros((1, d), jnp.float32).at[0, :depth + 1].set(alphas_raw)
    u2, dinv, dinv2, alphas = _prep_tc(dp.T, xp, araw_page)

    al16 = alphas[0, :_LANES]
    sraw = _poly_sc(u2, col3, row3, w3, dinv2.reshape(n_pad), al16,
                    n_pad, d, depth)
    xs3 = _post_tc(alphas, sraw, dinv)
    return jnp.concatenate(
        [x[:, None, :]] + [xs3[ld, :n, None, :] for ld in range(depth)],
        axis=1)
